# baseline (speedup 1.0000x reference)
import functools

import jax
import jax.numpy as jnp
from jax import lax
from jax.experimental import pallas as pl
from jax.experimental.pallas import tpu as pltpu
from jax.experimental.pallas import tpu_sc as plsc

_NUM_CORES = 2
_NUM_SUBCORES = 16
_NW = _NUM_CORES * _NUM_SUBCORES
_LANES = 16


def _make_agg_kernel(N, DEG, D, C):
    n_chunks = (N + C - 1) // C
    iters = (n_chunks + _NW - 1) // _NW
    groups = D // _LANES
    inv_deg = 1.0 / DEG

    mesh = plsc.VectorSubcoreMesh(core_axis_name="c", subcore_axis_name="s")

    @functools.partial(
        pl.kernel,
        out_type=jax.ShapeDtypeStruct((N, 2 * D), jnp.float32),
        mesh=mesh,
        scratch_types=[
            pltpu.VMEM((C, DEG, D), jnp.float32),
            pltpu.VMEM((C, 2 * D), jnp.float32),
        ],
    )
    def agg(mb_hbm, out_hbm, buf, obuf):
        wid = lax.axis_index("s") * _NUM_CORES + lax.axis_index("c")

        def chunk_body(k, _):
            cid = k * _NW + wid

            @pl.when(cid < n_chunks)
            def _():
                base = cid * C
                pltpu.sync_copy(mb_hbm.at[pl.ds(base, C)], buf)
                for n in range(C):
                    for g in range(groups):
                        col = pl.ds(g * _LANES, _LANES)
                        s = buf[n, 0, col]
                        m = s
                        for r in range(1, DEG):
                            v = buf[n, r, col]
                            s = s + v
                            m = jnp.maximum(m, v)
                        obuf[n, col] = s * inv_deg
                        obuf[n, pl.ds(D + g * _LANES, _LANES)] = m
                pltpu.sync_copy(obuf, out_hbm.at[pl.ds(base, C)])

            return 0

        lax.fori_loop(0, iters, chunk_body, 0)

    return agg


def _mm_body(a_ref, wt_ref, b_ref, o_ref):
    o_ref[...] = (
        jnp.dot(a_ref[...], wt_ref[...], preferred_element_type=jnp.float32)
        + b_ref[...]
    )


def kernel(mailbox, W, b):
    N, DEG, D = mailbox.shape
    C = 8

    agg_fn = _make_agg_kernel(N, DEG, D, C)
    agg = agg_fn(mailbox)

    Bn = 2000
    out = pl.pallas_call(
        _mm_body,
        grid=(N // Bn,),
        in_specs=[
            pl.BlockSpec((Bn, 2 * D), lambda i: (i, 0)),
            pl.BlockSpec((2 * D, D), lambda i: (0, 0)),
            pl.BlockSpec((1, D), lambda i: (0, 0)),
        ],
        out_specs=pl.BlockSpec((Bn, D), lambda i: (i, 0)),
        out_shape=jax.ShapeDtypeStruct((N, D), jnp.float32),
    )(agg, W.T, b.reshape(1, D))
    return out


# speedup vs baseline: 2.2268x; 2.2268x over previous
import functools

import jax
import jax.numpy as jnp
from jax import lax
from jax.experimental import pallas as pl
from jax.experimental.pallas import tpu as pltpu
from jax.experimental.pallas import tpu_sc as plsc

_NUM_CORES = 2
_NUM_SUBCORES = 16
_NW = _NUM_CORES * _NUM_SUBCORES
_LANES = 16


def _make_agg_kernel(N, DEG, D, C):
    assert N % C == 0
    n_chunks = N // C
    max_iters = (n_chunks + _NW - 1) // _NW
    assert max_iters % 2 == 0
    groups = D // _LANES
    inv_deg = 1.0 / DEG

    mesh = plsc.VectorSubcoreMesh(core_axis_name="c", subcore_axis_name="s")

    @functools.partial(
        pl.kernel,
        out_type=jax.ShapeDtypeStruct((N, 2 * D), jnp.float32),
        mesh=mesh,
        scratch_types=[
            pltpu.VMEM((2, C, DEG, D), jnp.float32),
            pltpu.VMEM((2, C, 2 * D), jnp.float32),
            pltpu.SemaphoreType.DMA,
            pltpu.SemaphoreType.DMA,
            pltpu.SemaphoreType.DMA,
            pltpu.SemaphoreType.DMA,
        ],
    )
    def agg(mb_hbm, out_hbm, buf, obuf, isem0, isem1, osem0, osem1):
        wid = lax.axis_index("s") * _NUM_CORES + lax.axis_index("c")
        my = (n_chunks - wid + _NW - 1) // _NW
        isems = (isem0, isem1)
        osems = (osem0, osem1)

        def in_copy(k, p):
            base = (k * _NW + wid) * C
            return pltpu.make_async_copy(
                mb_hbm.at[pl.ds(base, C)], buf.at[p], isems[p]
            )

        def out_copy(k, p):
            base = (k * _NW + wid) * C
            return pltpu.make_async_copy(
                obuf.at[p], out_hbm.at[pl.ds(base, C)], osems[p]
            )

        in_copy(0, 0).start()

        def body(kk, _):
            for p in range(2):
                k = kk * 2 + p
                np_ = 1 - p

                @pl.when(k + 1 < my)
                def _():
                    in_copy(k + 1, np_).start()

                @pl.when(k < my)
                def _():
                    in_copy(k, p).wait()

                    @pl.when(k >= 2)
                    def _():
                        out_copy(k - 2, p).wait()

                    def node_body(n, carry):
                        for g in range(groups):
                            col = pl.ds(g * _LANES, _LANES)
                            s = buf[p, n, 0, col]
                            m = s
                            for r in range(1, DEG):
                                v = buf[p, n, r, col]
                                s = s + v
                                m = jnp.maximum(m, v)
                            obuf[p, n, col] = s * inv_deg
                            obuf[p, n, pl.ds(D + g * _LANES, _LANES)] = m
                        return carry

                    lax.fori_loop(0, C, node_body, 0)
                    out_copy(k, p).start()

            return 0

        lax.fori_loop(0, max_iters // 2, body, 0)

        out_copy(0, 0).wait()
        out_copy(0, 1).wait()

    return agg


def _mm_body(a_ref, wt_ref, b_ref, o_ref):
    o_ref[...] = (
        jnp.dot(a_ref[...], wt_ref[...], preferred_element_type=jnp.float32)
        + b_ref[...]
    )


def kernel(mailbox, W, b):
    N, DEG, D = mailbox.shape
    C = 8

    agg_fn = _make_agg_kernel(N, DEG, D, C)
    agg = agg_fn(mailbox)

    Bn = 2000
    out = pl.pallas_call(
        _mm_body,
        grid=(N // Bn,),
        in_specs=[
            pl.BlockSpec((Bn, 2 * D), lambda i: (i, 0)),
            pl.BlockSpec((2 * D, D), lambda i: (0, 0)),
            pl.BlockSpec((1, D), lambda i: (0, 0)),
        ],
        out_specs=pl.BlockSpec((Bn, D), lambda i: (i, 0)),
        out_shape=jax.ShapeDtypeStruct((N, D), jnp.float32),
    )(agg, W.T, b.reshape(1, D))
    return out
